# baseline (speedup 1.0000x reference)
import functools

import jax
import jax.numpy as jnp
import numpy as np
from jax import lax
from jax.experimental import pallas as pl
from jax.experimental.pallas import tpu as pltpu
from jax.experimental.pallas import tpu_sc as plsc

NC = 2
NS = 16
NW = NC * NS

CHUNK = 128
LANES = 16


def _sc_aggregate(x, srcc, dstc, zacc, zcnt, o16, rows_pad, chunks_per_w,
                  extra):
    d = x.shape[1]
    rows_per_w = rows_pad // NS

    mesh = plsc.VectorSubcoreMesh(core_axis_name="c", subcore_axis_name="s")

    @functools.partial(
        pl.kernel,
        out_type=(
            jax.ShapeDtypeStruct((NC, rows_pad, d), jnp.float32),
            jax.ShapeDtypeStruct((NC, rows_pad, LANES), jnp.float32),
        ),
        mesh=mesh,
        compiler_params=pltpu.CompilerParams(use_tc_tiling_on_sc=False),
        scratch_types=[
            pltpu.VMEM((2, CHUNK), jnp.int32),
            pltpu.VMEM((2, CHUNK), jnp.int32),
            pltpu.VMEM((2, CHUNK), jnp.int32),
            pltpu.VMEM((2, CHUNK), jnp.int32),
            pltpu.VMEM((CHUNK, d), jnp.float32),
            pltpu.VMEM((CHUNK, d), jnp.float32),
            pltpu.VMEM((CHUNK, LANES), jnp.float32),
            pltpu.VMEM_SHARED((rows_pad, d), jnp.float32),
            pltpu.VMEM_SHARED((rows_pad, LANES), jnp.float32),
            pltpu.SemaphoreType.DMA,
            pltpu.SemaphoreType.DMA,
            pltpu.SemaphoreType.DMA,
            pltpu.SemaphoreType.DMA,
            pltpu.SemaphoreType.DMA,
            pltpu.SemaphoreType.DMA,
            pltpu.SemaphoreType.DMA,
        ],
    )
    def k(x_hbm, src_hbm, dst_hbm, zacc_hbm, zcnt_hbm, o16_hbm,
          parts_hbm, cnts_hbm, iA, iB, iC, iD, buf0, buf1, ones_v, acc, cnt,
          sem0, sem1, semA, semB, semC, semD, zsem):
        cid = lax.axis_index("c")
        sid = lax.axis_index("s")
        wid = sid * NC + cid
        cbase = wid * chunks_per_w

        def iload(j, ib, sem):
            pltpu.async_copy(src_hbm.at[j], ib.at[0], sem)
            pltpu.async_copy(dst_hbm.at[j], ib.at[1], sem)

        def iwait(ib, sem):
            pltpu.make_async_copy(src_hbm.at[0], ib.at[0], sem).wait()
            pltpu.make_async_copy(dst_hbm.at[0], ib.at[1], sem).wait()

        def gather(ib, buf, sem):
            pltpu.async_copy(x_hbm.at[ib.at[0]], buf, sem)

        def gwait(ib, buf, sem):
            pltpu.make_async_copy(x_hbm.at[ib.at[0]], buf, sem).wait()

        def scatter(ib, buf):
            pltpu.sync_copy(buf, acc.at[ib.at[1]], add=True)
            pltpu.sync_copy(ones_v, cnt.at[ib.at[1]], add=True)

        iload(cbase + 0, iA, semA)
        iload(cbase + 1, iB, semB)
        iload(cbase + 2, iC, semC)
        iload(cbase + 3, iD, semD)
        iwait(iA, semA)
        gather(iA, buf0, sem0)

        row0 = pl.multiple_of(sid * rows_per_w, 8)
        zpairs = [
            (zacc_hbm.at[pl.ds(0, rows_per_w)],
             acc.at[pl.ds(row0, rows_per_w)]),
            (zcnt_hbm.at[pl.ds(0, rows_per_w)],
             cnt.at[pl.ds(row0, rows_per_w)]),
            (o16_hbm, ones_v),
        ]
        for s, dref in zpairs:
            pltpu.async_copy(s, dref, zsem)
        for s, dref in zpairs:
            pltpu.make_async_copy(s, dref, zsem).wait()
        plsc.subcore_barrier()

        nquads, r = divmod(chunks_per_w, 4)

        def body(g, carry):
            q = cbase + g * 4
            iwait(iB, semB)
            gather(iB, buf1, sem1)
            gwait(iA, buf0, sem0)
            scatter(iA, buf0)
            iload(q + 4, iA, semA)
            iwait(iC, semC)
            gather(iC, buf0, sem0)
            gwait(iB, buf1, sem1)
            scatter(iB, buf1)
            iload(q + 5, iB, semB)
            iwait(iD, semD)
            gather(iD, buf1, sem1)
            gwait(iC, buf0, sem0)
            scatter(iC, buf0)
            iload(q + 6, iC, semC)
            iwait(iA, semA)
            gather(iA, buf0, sem0)
            gwait(iD, buf1, sem1)
            scatter(iD, buf1)
            iload(q + 7, iD, semD)
            return carry

        lax.fori_loop(0, nquads, body, 0)

        if r == 0:
            gwait(iA, buf0, sem0)
            iwait(iB, semB)
            iwait(iC, semC)
            iwait(iD, semD)
        elif r == 1:
            gwait(iA, buf0, sem0)
            scatter(iA, buf0)
            iwait(iB, semB)
            iwait(iC, semC)
            iwait(iD, semD)
        elif r == 2:
            iwait(iB, semB)
            gather(iB, buf1, sem1)
            gwait(iA, buf0, sem0)
            scatter(iA, buf0)
            gwait(iB, buf1, sem1)
            scatter(iB, buf1)
            iwait(iC, semC)
            iwait(iD, semD)
        else:
            iwait(iB, semB)
            gather(iB, buf1, sem1)
            gwait(iA, buf0, sem0)
            scatter(iA, buf0)
            iwait(iC, semC)
            gather(iC, buf0, sem0)
            gwait(iB, buf1, sem1)
            scatter(iB, buf1)
            gwait(iC, buf0, sem0)
            scatter(iC, buf0)
            iwait(iD, semD)

        if extra:
            @pl.when(wid < extra)
            def _():
                jx = NW * chunks_per_w + wid
                iload(jx, iA, semA)
                iwait(iA, semA)
                gather(iA, buf0, sem0)
                gwait(iA, buf0, sem0)
                scatter(iA, buf0)

        plsc.subcore_barrier()

        pltpu.sync_copy(
            acc.at[pl.ds(row0, rows_per_w)],
            parts_hbm.at[cid, pl.ds(row0, rows_per_w)],
        )
        pltpu.sync_copy(
            cnt.at[pl.ds(row0, rows_per_w)],
            cnts_hbm.at[cid, pl.ds(row0, rows_per_w)],
        )

    return k(x, srcc, dstc, zacc, zcnt, o16)


def _tc_combine(summed, cnts, x, W_l, b_l, W_r, n, d_in, d_out):
    blk = 2000
    grid = -(-n // blk)
    b_l2 = b_l.reshape(1, d_out)

    def body(p_ref, c_ref, x_ref, wl_ref, bl_ref, wr_ref, o_ref):
        cnt = jnp.maximum(c_ref[0, :, 0:1] + c_ref[1, :, 0:1], 1.0)
        mean = p_ref[...] / cnt
        o_ref[...] = (
            lax.dot_general(mean, wl_ref[...], (((1,), (1,)), ((), ())),
                            preferred_element_type=jnp.float32)
            + bl_ref[...]
            + lax.dot_general(x_ref[...], wr_ref[...], (((1,), (1,)), ((), ())),
                              preferred_element_type=jnp.float32)
        )

    return pl.pallas_call(
        body,
        grid=(grid,),
        in_specs=[
            pl.BlockSpec((blk, d_in), lambda i: (i, 0)),
            pl.BlockSpec((NC, blk, LANES), lambda i: (0, i, 0)),
            pl.BlockSpec((blk, d_in), lambda i: (i, 0)),
            pl.BlockSpec((d_out, d_in), lambda i: (0, 0)),
            pl.BlockSpec((1, d_out), lambda i: (0, 0)),
            pl.BlockSpec((d_out, d_in), lambda i: (0, 0)),
        ],
        out_specs=pl.BlockSpec((blk, d_out), lambda i: (i, 0)),
        out_shape=jax.ShapeDtypeStruct((n, d_out), jnp.float32),
    )(summed, cnts, x, W_l, b_l2, W_r)


def kernel(x, edge_index, W_l, b_l, W_r):
    n, d_in = x.shape
    d_out = W_l.shape[0]
    e = edge_index.shape[1]

    rows_pad = ((n + LANES + NS * 8 - 1) // (NS * 8)) * (NS * 8)

    ei32 = edge_index.astype(jnp.int32)
    total_chunks = e // CHUNK
    if e % CHUNK == 0 and total_chunks % NW >= 4:
        srcc = ei32[0].reshape(total_chunks, CHUNK)
        dstc = ei32[1].reshape(total_chunks, CHUNK)
    else:
        total_chunks = -(-e // CHUNK)
        if total_chunks % NW < 4:
            total_chunks += 4 - total_chunks % NW
        e_pad = total_chunks * CHUNK
        pad_ar = jnp.arange(e_pad - e, dtype=jnp.int32)
        srcc = jnp.concatenate([ei32[0], pad_ar % n]).reshape(-1, CHUNK)
        dstc = jnp.concatenate([ei32[1], n + (pad_ar % LANES)]).reshape(-1, CHUNK)
    chunks_per_w, extra = divmod(total_chunks, NW)

    rows_per_w = rows_pad // NS
    zacc = np.zeros((rows_per_w, d_in), np.float32)
    zcnt = np.zeros((rows_per_w, LANES), np.float32)
    o16 = np.ones((CHUNK, LANES), np.float32)
    parts, cnts = _sc_aggregate(x, srcc, dstc, zacc, zcnt, o16, rows_pad,
                                chunks_per_w, extra)
    summed = parts[0] + parts[1]
    return _tc_combine(summed, cnts, x, W_l, b_l, W_r, n, d_in, d_out)


# speedup vs baseline: 1.0873x; 1.0873x over previous
import functools

import jax
import jax.numpy as jnp
import numpy as np
from jax import lax
from jax.experimental import pallas as pl
from jax.experimental.pallas import tpu as pltpu
from jax.experimental.pallas import tpu_sc as plsc

NC = 2
NS = 16
NW = NC * NS

CHUNK = 128
LANES = 16


def _sc_aggregate(x, ei, zacc, zcnt, o16, rows_pad, chunks_per_w, extra):
    d = x.shape[1]
    rows_per_w = rows_pad // NS

    mesh = plsc.VectorSubcoreMesh(core_axis_name="c", subcore_axis_name="s")

    @functools.partial(
        pl.kernel,
        out_type=(
            jax.ShapeDtypeStruct((NC, rows_pad, d), jnp.float32),
            jax.ShapeDtypeStruct((NC, rows_pad, LANES), jnp.float32),
        ),
        mesh=mesh,
        compiler_params=pltpu.CompilerParams(use_tc_tiling_on_sc=False),
        scratch_types=[
            pltpu.VMEM((2, CHUNK), jnp.int32),
            pltpu.VMEM((2, CHUNK), jnp.int32),
            pltpu.VMEM((2, CHUNK), jnp.int32),
            pltpu.VMEM((2, CHUNK), jnp.int32),
            pltpu.VMEM((CHUNK, d), jnp.float32),
            pltpu.VMEM((CHUNK, d), jnp.float32),
            pltpu.VMEM((CHUNK, LANES), jnp.float32),
            pltpu.VMEM_SHARED((rows_pad, d), jnp.float32),
            pltpu.VMEM_SHARED((rows_pad, LANES), jnp.float32),
            pltpu.SemaphoreType.DMA,
            pltpu.SemaphoreType.DMA,
            pltpu.SemaphoreType.DMA,
            pltpu.SemaphoreType.DMA,
            pltpu.SemaphoreType.DMA,
            pltpu.SemaphoreType.DMA,
        ],
    )
    def k(x_hbm, ei_hbm, zacc_hbm, zcnt_hbm, o16_hbm, parts_hbm, cnts_hbm,
          iA, iB, iC, iD, buf0, buf1, ones_v, acc, cnt,
          sem0, sem1, semA, semB, semC, semD):
        cid = lax.axis_index("c")
        sid = lax.axis_index("s")
        wid = sid * NC + cid
        cbase = wid * chunks_per_w

        def iload(j, ib, sem):
            pltpu.async_copy(ei_hbm.at[0, j], ib.at[0], sem)
            pltpu.async_copy(ei_hbm.at[1, j], ib.at[1], sem)

        def iwait(ib, sem):
            pltpu.make_async_copy(ei_hbm.at[0, 0], ib.at[0], sem).wait()
            pltpu.make_async_copy(ei_hbm.at[1, 0], ib.at[1], sem).wait()

        def gather(ib, buf, sem):
            pltpu.async_copy(x_hbm.at[ib.at[0]], buf, sem)

        def gwait(ib, buf, sem):
            pltpu.make_async_copy(x_hbm.at[ib.at[0]], buf, sem).wait()

        def scatter(ib, buf):
            pltpu.sync_copy(buf, acc.at[ib.at[1]], add=True)
            pltpu.sync_copy(ones_v, cnt.at[ib.at[1]], add=True)

        iload(cbase + 0, iA, semA)
        iload(cbase + 1, iB, semB)
        iload(cbase + 2, iC, semC)
        iload(cbase + 3, iD, semD)
        iwait(iA, semA)
        gather(iA, buf0, sem0)

        row0 = pl.multiple_of(sid * rows_per_w, 8)
        pltpu.sync_copy(zacc_hbm.at[pl.ds(row0, rows_per_w)],
                        acc.at[pl.ds(row0, rows_per_w)])
        pltpu.sync_copy(zcnt_hbm.at[pl.ds(row0, rows_per_w)],
                        cnt.at[pl.ds(row0, rows_per_w)])
        pltpu.sync_copy(o16_hbm, ones_v)
        plsc.subcore_barrier()

        nquads, r = divmod(chunks_per_w, 4)

        def body(g, carry):
            q = cbase + g * 4
            iwait(iB, semB)
            gather(iB, buf1, sem1)
            gwait(iA, buf0, sem0)
            scatter(iA, buf0)
            iload(q + 4, iA, semA)
            iwait(iC, semC)
            gather(iC, buf0, sem0)
            gwait(iB, buf1, sem1)
            scatter(iB, buf1)
            iload(q + 5, iB, semB)
            iwait(iD, semD)
            gather(iD, buf1, sem1)
            gwait(iC, buf0, sem0)
            scatter(iC, buf0)
            iload(q + 6, iC, semC)
            iwait(iA, semA)
            gather(iA, buf0, sem0)
            gwait(iD, buf1, sem1)
            scatter(iD, buf1)
            iload(q + 7, iD, semD)
            return carry

        lax.fori_loop(0, nquads, body, 0)

        if r == 0:
            gwait(iA, buf0, sem0)
            iwait(iB, semB)
            iwait(iC, semC)
            iwait(iD, semD)
        elif r == 1:
            gwait(iA, buf0, sem0)
            scatter(iA, buf0)
            iwait(iB, semB)
            iwait(iC, semC)
            iwait(iD, semD)
        elif r == 2:
            iwait(iB, semB)
            gather(iB, buf1, sem1)
            gwait(iA, buf0, sem0)
            scatter(iA, buf0)
            gwait(iB, buf1, sem1)
            scatter(iB, buf1)
            iwait(iC, semC)
            iwait(iD, semD)
        else:
            iwait(iB, semB)
            gather(iB, buf1, sem1)
            gwait(iA, buf0, sem0)
            scatter(iA, buf0)
            iwait(iC, semC)
            gather(iC, buf0, sem0)
            gwait(iB, buf1, sem1)
            scatter(iB, buf1)
            gwait(iC, buf0, sem0)
            scatter(iC, buf0)
            iwait(iD, semD)

        if extra:
            @pl.when(wid < extra)
            def _():
                jx = NW * chunks_per_w + wid
                iload(jx, iA, semA)
                iwait(iA, semA)
                gather(iA, buf0, sem0)
                gwait(iA, buf0, sem0)
                scatter(iA, buf0)

        plsc.subcore_barrier()

        pltpu.sync_copy(
            acc.at[pl.ds(row0, rows_per_w)],
            parts_hbm.at[cid, pl.ds(row0, rows_per_w)],
        )
        pltpu.sync_copy(
            cnt.at[pl.ds(row0, rows_per_w)],
            cnts_hbm.at[cid, pl.ds(row0, rows_per_w)],
        )

    return k(x, ei, zacc, zcnt, o16)


def _tc_combine(parts, cnts, x, W_l, b_l, W_r, n, d_in, d_out):
    blk = 2000
    grid = -(-n // blk)
    b_l2 = b_l.reshape(1, d_out)

    def body(p_ref, c_ref, x_ref, wl_ref, bl_ref, wr_ref, o_ref):
        summed = p_ref[0] + p_ref[1]
        cnt = jnp.maximum(c_ref[0, :, 0:1] + c_ref[1, :, 0:1], 1.0)
        mean = summed / cnt
        o_ref[...] = (
            lax.dot_general(mean, wl_ref[...], (((1,), (1,)), ((), ())),
                            preferred_element_type=jnp.float32)
            + bl_ref[...]
            + lax.dot_general(x_ref[...], wr_ref[...], (((1,), (1,)), ((), ())),
                              preferred_element_type=jnp.float32)
        )

    return pl.pallas_call(
        body,
        grid=(grid,),
        in_specs=[
            pl.BlockSpec((NC, blk, d_in), lambda i: (0, i, 0)),
            pl.BlockSpec((NC, blk, LANES), lambda i: (0, i, 0)),
            pl.BlockSpec((blk, d_in), lambda i: (i, 0)),
            pl.BlockSpec((d_out, d_in), lambda i: (0, 0)),
            pl.BlockSpec((1, d_out), lambda i: (0, 0)),
            pl.BlockSpec((d_out, d_in), lambda i: (0, 0)),
        ],
        out_specs=pl.BlockSpec((blk, d_out), lambda i: (i, 0)),
        out_shape=jax.ShapeDtypeStruct((n, d_out), jnp.float32),
    )(parts, cnts, x, W_l, b_l2, W_r)


def kernel(x, edge_index, W_l, b_l, W_r):
    n, d_in = x.shape
    d_out = W_l.shape[0]
    e = edge_index.shape[1]

    rows_pad = ((n + LANES + NS * 8 - 1) // (NS * 8)) * (NS * 8)

    ei32 = edge_index.astype(jnp.int32)
    total_chunks = e // CHUNK
    if e % CHUNK == 0 and total_chunks % NW >= 4:
        ei = ei32.reshape(2, total_chunks, CHUNK)
    else:
        total_chunks = -(-e // CHUNK)
        if total_chunks % NW < 4:
            total_chunks += 4 - total_chunks % NW
        e_pad = total_chunks * CHUNK
        pad_ar = jnp.arange(e_pad - e, dtype=jnp.int32)
        src = jnp.concatenate([ei32[0], pad_ar % n])
        dst = jnp.concatenate([ei32[1], n + (pad_ar % LANES)])
        ei = jnp.stack([src, dst]).reshape(2, total_chunks, CHUNK)
    chunks_per_w, extra = divmod(total_chunks, NW)

    zacc = np.zeros((rows_pad, d_in), np.float32)
    zcnt = np.zeros((rows_pad, LANES), np.float32)
    o16 = np.ones((CHUNK, LANES), np.float32)
    parts, cnts = _sc_aggregate(x, ei, zacc, zcnt, o16, rows_pad,
                                chunks_per_w, extra)
    return _tc_combine(parts, cnts, x, W_l, b_l, W_r, n, d_in, d_out)
